# Initial kernel scaffold; baseline (speedup 1.0000x reference)
import jax
import jax.numpy as jnp
from jax.experimental import pallas as pl


def kernel(x, edge_index, W1, b1, W2, b2, W3, b3, W4, b4, Wp, bp):
    raise NotImplementedError("write your pallas kernel here")


# speedup vs baseline: 5.4357x; 5.4357x over previous
import functools

import jax
import jax.numpy as jnp
from jax import lax
from jax.experimental import pallas as pl
from jax.experimental.pallas import tpu as pltpu
from jax.experimental.pallas import tpu_sc as plsc

N = 10000
E = 320000
NC, NS, L = 2, 16, 16
NT = NC * NS
B = 256
NB = 40
EPT = NB * B
E_PAD = NT * EPT
NA = N + 8
RPT = 624
REM = N - NS * RPT

_MESH = plsc.VectorSubcoreMesh(core_axis_name="c", subcore_axis_name="s")
_HIGH = jax.lax.Precision.HIGHEST


def _leaky(v):
    return jnp.where(v >= 0, v, 0.01 * v)


def _deg_body(dst_hbm, ones_hbm, zero_hbm, out_hbm, didx_b, ones_v, acc_sh, sem):
    c = lax.axis_index("c")
    s = lax.axis_index("s")
    tid = c * NS + s
    pltpu.sync_copy(ones_hbm, ones_v)
    pltpu.sync_copy(zero_hbm, acc_sh.at[pl.ds(s * RPT, RPT)])

    @pl.when(s == 0)
    def _():
        pltpu.sync_copy(zero_hbm.at[pl.ds(0, REM)],
                        acc_sh.at[pl.ds(NS * RPT, REM)])

    plsc.subcore_barrier()

    def body(i, _):
        pltpu.sync_copy(dst_hbm.at[tid, i], didx_b)
        pltpu.sync_copy(ones_v, acc_sh.at[didx_b], add=True)
        return 0

    lax.fori_loop(0, NB, body, 0)
    plsc.subcore_barrier()
    pltpu.sync_copy(acc_sh.at[pl.ds(s * RPT, RPT)],
                    out_hbm.at[c, pl.ds(s * RPT, RPT)])

    @pl.when(s == 0)
    def _():
        pltpu.sync_copy(acc_sh.at[pl.ds(NS * RPT, REM)],
                        out_hbm.at[c, pl.ds(NS * RPT, REM)])


@functools.partial(
    pl.kernel,
    out_type=jax.ShapeDtypeStruct((NC, N, 128), jnp.float32),
    mesh=_MESH,
    scratch_types=[
        pltpu.VMEM((B,), jnp.int32),
        pltpu.VMEM((B, 128), jnp.float32),
        pltpu.VMEM_SHARED((NA, 128), jnp.float32),
        pltpu.SemaphoreType.DMA,
    ],
)
def _deg_kernel(dst_hbm, ones_hbm, zero_hbm, out_hbm, didx_b, ones_v, acc_sh, sem):
    _deg_body(dst_hbm, ones_hbm, zero_hbm, out_hbm, didx_b, ones_v, acc_sh, sem)


def _agg_body(W, table_hbm, src_hbm, dst_hbm, zero_hbm, out_hbm,
              sidx_b, didx_b, rows, acc_sh, sem):
    c = lax.axis_index("c")
    s = lax.axis_index("s")
    tid = c * NS + s
    pltpu.sync_copy(zero_hbm, acc_sh.at[pl.ds(s * RPT, RPT)])

    @pl.when(s == 0)
    def _():
        pltpu.sync_copy(zero_hbm.at[pl.ds(0, REM)],
                        acc_sh.at[pl.ds(NS * RPT, REM)])

    plsc.subcore_barrier()

    def body(i, _):
        pltpu.sync_copy(src_hbm.at[tid, i], sidx_b)
        pltpu.sync_copy(dst_hbm.at[tid, i], didx_b)
        pltpu.async_copy(table_hbm.at[sidx_b], rows, sem).wait()
        pltpu.sync_copy(rows, acc_sh.at[didx_b], add=True)
        return 0

    lax.fori_loop(0, NB, body, 0)
    plsc.subcore_barrier()
    pltpu.sync_copy(acc_sh.at[pl.ds(s * RPT, RPT)],
                    out_hbm.at[c, pl.ds(s * RPT, RPT)])

    @pl.when(s == 0)
    def _():
        pltpu.sync_copy(acc_sh.at[pl.ds(NS * RPT, REM)],
                        out_hbm.at[c, pl.ds(NS * RPT, REM)])


@functools.cache
def _make_agg(W):
    @functools.partial(
        pl.kernel,
        out_type=jax.ShapeDtypeStruct((NC, N, W), jnp.float32),
        mesh=_MESH,
        scratch_types=[
            pltpu.VMEM((B,), jnp.int32),
            pltpu.VMEM((B,), jnp.int32),
            pltpu.VMEM((B, W), jnp.float32),
            pltpu.VMEM_SHARED((NA, W), jnp.float32),
            pltpu.SemaphoreType.DMA,
        ],
    )
    def _agg(table_hbm, src_hbm, dst_hbm, zero_hbm, out_hbm,
             sidx_b, didx_b, rows, acc_sh, sem):
        _agg_body(W, table_hbm, src_hbm, dst_hbm, zero_hbm, out_hbm,
                  sidx_b, didx_b, rows, acc_sh, sem)

    return _agg


def _tca_body(dp_ref, x_ref, dinv_ref, xp_ref):
    deg = dp_ref[0, :, 0:1] + dp_ref[1, :, 0:1] + 1.0
    dinv = lax.rsqrt(deg)
    dinv_ref[...] = dinv
    xp_ref[...] = x_ref[...] * dinv


def _tc_a(deg_parts, x):
    return pl.pallas_call(
        _tca_body,
        out_shape=(jax.ShapeDtypeStruct((N, 1), jnp.float32),
                   jax.ShapeDtypeStruct((N, 128), jnp.float32)),
    )(deg_parts, x)


_RB = 1000
_G = N // _RB


def _row_spec(w):
    return pl.BlockSpec((_RB, w), lambda i: (i, 0))


def _part_spec(w):
    return pl.BlockSpec((NC, _RB, w), lambda i: (0, i, 0))


def _full_spec(a, b):
    return pl.BlockSpec((a, b), lambda i: (0, 0))


def _tcb_body(t0_ref, xp_ref, dinv_ref, w1_ref, b1_ref, ga_ref, gb_ref):
    dinv = dinv_ref[...]
    s0 = dinv * (t0_ref[0] + t0_ref[1] + xp_ref[...])
    h1 = _leaky(jnp.dot(s0, w1_ref[...], precision=_HIGH,
                        preferred_element_type=jnp.float32) + b1_ref[...])
    g1 = dinv * h1
    ga_ref[...] = g1[:, :128]
    gb_ref[...] = g1[:, 128:]


def _tc_b(t0p, xp, dinv, W1, b1):
    return pl.pallas_call(
        _tcb_body,
        grid=(_G,),
        in_specs=[_part_spec(128), _row_spec(128), _row_spec(1),
                  _full_spec(128, 256), _full_spec(1, 256)],
        out_specs=(_row_spec(128), _row_spec(128)),
        out_shape=(jax.ShapeDtypeStruct((N, 128), jnp.float32),
                   jax.ShapeDtypeStruct((N, 128), jnp.float32)),
    )(t0p, xp, dinv, W1, b1.reshape(1, 256))


def _tcc_body(ta_ref, tb_ref, ga_ref, gb_ref, dinv_ref, w2_ref, b2_ref,
              w3_ref, g2_ref):
    dinv = dinv_ref[...]
    sa = dinv * (ta_ref[0] + ta_ref[1] + ga_ref[...])
    sb = dinv * (tb_ref[0] + tb_ref[1] + gb_ref[...])
    s1 = jnp.concatenate([sa, sb], axis=1)
    h2 = _leaky(jnp.dot(s1, w2_ref[...], precision=_HIGH,
                        preferred_element_type=jnp.float32) + b2_ref[...])
    g2_ref[...] = dinv * jnp.dot(h2, w3_ref[...], precision=_HIGH,
                                 preferred_element_type=jnp.float32)


def _tc_c(t1ap, t1bp, g1a, g1b, dinv, W2, b2, W3):
    return pl.pallas_call(
        _tcc_body,
        grid=(_G,),
        in_specs=[_part_spec(128), _part_spec(128), _row_spec(128),
                  _row_spec(128), _row_spec(1), _full_spec(256, 256),
                  _full_spec(1, 256), _full_spec(256, 128)],
        out_specs=_row_spec(128),
        out_shape=jax.ShapeDtypeStruct((N, 128), jnp.float32),
    )(t1ap, t1bp, g1a, g1b, dinv, W2, b2.reshape(1, 256), W3)


def _tcd_body(t2_ref, g2_ref, dinv_ref, b3_ref, g3_ref):
    dinv = dinv_ref[...]
    h3 = _leaky(dinv * (t2_ref[0] + t2_ref[1] + g2_ref[...]) + b3_ref[...])
    g3_ref[...] = dinv * h3


def _tc_d(t2p, g2, dinv, b3):
    return pl.pallas_call(
        _tcd_body,
        grid=(_G,),
        in_specs=[_part_spec(128), _row_spec(128), _row_spec(1),
                  _full_spec(1, 128)],
        out_specs=_row_spec(128),
        out_shape=jax.ShapeDtypeStruct((N, 128), jnp.float32),
    )(t2p, g2, dinv, b3.reshape(1, 128))


def _tce_body(t3_ref, g3_ref, dinv_ref, w4_ref, b4_ref, wp_ref, bp_ref,
              out_ref):
    dinv = dinv_ref[...]
    s3 = dinv * (t3_ref[0] + t3_ref[1] + g3_ref[...])
    h4 = _leaky(jnp.dot(s3, w4_ref[...], precision=_HIGH,
                        preferred_element_type=jnp.float32) + b4_ref[...])
    out_ref[...] = jnp.dot(h4, wp_ref[...], precision=_HIGH,
                           preferred_element_type=jnp.float32) + bp_ref[...]


def _tc_e(t3p, g3, dinv, W4, b4, Wp, bp):
    return pl.pallas_call(
        _tce_body,
        grid=(_G,),
        in_specs=[_part_spec(128), _row_spec(128), _row_spec(1),
                  _full_spec(128, 64), _full_spec(1, 64), _full_spec(64, 40),
                  _full_spec(1, 40)],
        out_specs=_row_spec(40),
        out_shape=jax.ShapeDtypeStruct((N, 40), jnp.float32),
    )(t3p, g3, dinv, W4, b4.reshape(1, 64), Wp, bp.reshape(1, 40))


def kernel(x, edge_index, W1, b1, W2, b2, W3, b3, W4, b4, Wp, bp):
    ei = edge_index.astype(jnp.int32)
    pad = E_PAD - E
    srcp = jnp.concatenate([ei[0], jnp.zeros((pad,), jnp.int32)])
    dstp = jnp.concatenate([ei[1], jnp.full((pad,), N, jnp.int32)])
    src3 = srcp.reshape(NT, NB, B)
    dst3 = dstp.reshape(NT, NB, B)
    ones128 = jnp.ones((B, 128), jnp.float32)
    z128 = jnp.zeros((RPT, 128), jnp.float32)

    agg128 = _make_agg(128)

    deg_parts = _deg_kernel(dst3, ones128, z128)
    dinv, xp = _tc_a(deg_parts, x)

    t0p = agg128(xp, src3, dst3, z128)
    g1a, g1b = _tc_b(t0p, xp, dinv, W1, b1)

    t1ap = agg128(g1a, src3, dst3, z128)
    t1bp = agg128(g1b, src3, dst3, z128)
    g2 = _tc_c(t1ap, t1bp, g1a, g1b, dinv, W2, b2, W3)

    t2p = agg128(g2, src3, dst3, z128)
    g3 = _tc_d(t2p, g2, dinv, b3)

    t3p = agg128(g3, src3, dst3, z128)
    return _tc_e(t3p, g3, dinv, W4, b4, Wp, bp)


# speedup vs baseline: 5.7546x; 1.0587x over previous
import functools

import jax
import jax.numpy as jnp
from jax import lax
from jax.experimental import pallas as pl
from jax.experimental.pallas import tpu as pltpu
from jax.experimental.pallas import tpu_sc as plsc

N = 10000
E = 320000
NC, NS, L = 2, 16, 16
NT = NC * NS
B = 128
NB = 80
EPT = NB * B
E_PAD = NT * EPT
NA = N + 8
RPT = 624
REM = N - NS * RPT

_MESH = plsc.VectorSubcoreMesh(core_axis_name="c", subcore_axis_name="s")
_HIGH = jax.lax.Precision.HIGHEST


def _leaky(v):
    return jnp.where(v >= 0, v, 0.01 * v)


def _deg_body(dst_hbm, ones_hbm, zero_hbm, out_hbm, didx_b, ones_v, acc_sh, sem):
    c = lax.axis_index("c")
    s = lax.axis_index("s")
    tid = c * NS + s
    pltpu.sync_copy(ones_hbm, ones_v)
    pltpu.sync_copy(zero_hbm, acc_sh.at[pl.ds(s * RPT, RPT)])

    @pl.when(s == 0)
    def _():
        pltpu.sync_copy(zero_hbm.at[pl.ds(0, REM)],
                        acc_sh.at[pl.ds(NS * RPT, REM)])

    plsc.subcore_barrier()

    def body(i, _):
        pltpu.sync_copy(dst_hbm.at[tid, i], didx_b)
        pltpu.sync_copy(ones_v, acc_sh.at[didx_b], add=True)
        return 0

    lax.fori_loop(0, NB, body, 0)
    plsc.subcore_barrier()
    pltpu.sync_copy(acc_sh.at[pl.ds(s * RPT, RPT)],
                    out_hbm.at[c, pl.ds(s * RPT, RPT)])

    @pl.when(s == 0)
    def _():
        pltpu.sync_copy(acc_sh.at[pl.ds(NS * RPT, REM)],
                        out_hbm.at[c, pl.ds(NS * RPT, REM)])


@functools.partial(
    pl.kernel,
    out_type=jax.ShapeDtypeStruct((NC, N, 128), jnp.float32),
    mesh=_MESH,
    scratch_types=[
        pltpu.VMEM((B,), jnp.int32),
        pltpu.VMEM((B, 128), jnp.float32),
        pltpu.VMEM_SHARED((NA, 128), jnp.float32),
        pltpu.SemaphoreType.DMA,
    ],
)
def _deg_kernel(dst_hbm, ones_hbm, zero_hbm, out_hbm, didx_b, ones_v, acc_sh, sem):
    _deg_body(dst_hbm, ones_hbm, zero_hbm, out_hbm, didx_b, ones_v, acc_sh, sem)


def _agg_body(W, table_hbm, src_hbm, dst_hbm, zero_hbm, out_hbm,
              sidx, didx, rows, acc_sh, isem, gsem, ssem):
    c = lax.axis_index("c")
    s = lax.axis_index("s")
    tid = c * NS + s
    pltpu.sync_copy(zero_hbm, acc_sh.at[pl.ds(s * RPT, RPT)])

    @pl.when(s == 0)
    def _():
        pltpu.sync_copy(zero_hbm.at[pl.ds(0, REM)],
                        acc_sh.at[pl.ds(NS * RPT, REM)])

    for q in range(2):
        pltpu.async_copy(src_hbm.at[tid, q], sidx[q], isem[q])
        pltpu.async_copy(dst_hbm.at[tid, q], didx[q], isem[q])

    plsc.subcore_barrier()

    def body(j, _):
        for p in range(4):
            b = 4 * j + p
            q, r = p, p % 2
            pltpu.make_async_copy(src_hbm.at[tid, 0], sidx[q], isem[q]).wait()
            pltpu.make_async_copy(dst_hbm.at[tid, 0], didx[q], isem[q]).wait()
            if p < 2:
                @pl.when(j > 0)
                def _():
                    pltpu.make_async_copy(
                        rows[r], acc_sh.at[pl.ds(0, B)], ssem[r]).wait()
            else:
                pltpu.make_async_copy(
                    rows[r], acc_sh.at[pl.ds(0, B)], ssem[r]).wait()
            nb = b + 2
            nb = jnp.where(nb < NB, nb, nb - NB)
            nq = (p + 2) % 4
            pltpu.async_copy(src_hbm.at[tid, nb], sidx[nq], isem[nq])
            pltpu.async_copy(dst_hbm.at[tid, nb], didx[nq], isem[nq])
            pltpu.async_copy(table_hbm.at[sidx[q]], rows[r], gsem).wait()
            pltpu.async_copy(rows[r], acc_sh.at[didx[q]], ssem[r], add=True)
        return 0

    lax.fori_loop(0, NB // 4, body, 0)
    for q in range(2):
        pltpu.make_async_copy(src_hbm.at[tid, 0], sidx[q], isem[q]).wait()
        pltpu.make_async_copy(dst_hbm.at[tid, 0], didx[q], isem[q]).wait()
    for r in range(2):
        pltpu.make_async_copy(rows[r], acc_sh.at[pl.ds(0, B)], ssem[r]).wait()
    plsc.subcore_barrier()
    pltpu.sync_copy(acc_sh.at[pl.ds(s * RPT, RPT)],
                    out_hbm.at[c, pl.ds(s * RPT, RPT)])

    @pl.when(s == 0)
    def _():
        pltpu.sync_copy(acc_sh.at[pl.ds(NS * RPT, REM)],
                        out_hbm.at[c, pl.ds(NS * RPT, REM)])


@functools.cache
def _make_agg(W):
    @functools.partial(
        pl.kernel,
        out_type=jax.ShapeDtypeStruct((NC, N, W), jnp.float32),
        mesh=_MESH,
        scratch_types=[
            [pltpu.VMEM((B,), jnp.int32)] * 4,
            [pltpu.VMEM((B,), jnp.int32)] * 4,
            [pltpu.VMEM((B, W), jnp.float32)] * 2,
            pltpu.VMEM_SHARED((NA, W), jnp.float32),
            [pltpu.SemaphoreType.DMA] * 4,
            pltpu.SemaphoreType.DMA,
            [pltpu.SemaphoreType.DMA] * 2,
        ],
    )
    def _agg(table_hbm, src_hbm, dst_hbm, zero_hbm, out_hbm,
             sidx, didx, rows, acc_sh, isem, gsem, ssem):
        _agg_body(W, table_hbm, src_hbm, dst_hbm, zero_hbm, out_hbm,
                  sidx, didx, rows, acc_sh, isem, gsem, ssem)

    return _agg


def _tca_body(dp_ref, x_ref, dinv_ref, xp_ref):
    deg = dp_ref[0, :, 0:1] + dp_ref[1, :, 0:1] + 1.0
    dinv = lax.rsqrt(deg)
    dinv_ref[...] = dinv
    xp_ref[...] = x_ref[...] * dinv


def _tc_a(deg_parts, x):
    return pl.pallas_call(
        _tca_body,
        out_shape=(jax.ShapeDtypeStruct((N, 1), jnp.float32),
                   jax.ShapeDtypeStruct((N, 128), jnp.float32)),
    )(deg_parts, x)


_RB = 1000
_G = N // _RB


def _row_spec(w):
    return pl.BlockSpec((_RB, w), lambda i: (i, 0))


def _part_spec(w):
    return pl.BlockSpec((NC, _RB, w), lambda i: (0, i, 0))


def _full_spec(a, b):
    return pl.BlockSpec((a, b), lambda i: (0, 0))


def _tcb_body(t0_ref, xp_ref, dinv_ref, w1_ref, b1_ref, ga_ref, gb_ref):
    dinv = dinv_ref[...]
    s0 = dinv * (t0_ref[0] + t0_ref[1] + xp_ref[...])
    h1 = _leaky(jnp.dot(s0, w1_ref[...], precision=_HIGH,
                        preferred_element_type=jnp.float32) + b1_ref[...])
    g1 = dinv * h1
    ga_ref[...] = g1[:, :128]
    gb_ref[...] = g1[:, 128:]


def _tc_b(t0p, xp, dinv, W1, b1):
    return pl.pallas_call(
        _tcb_body,
        grid=(_G,),
        in_specs=[_part_spec(128), _row_spec(128), _row_spec(1),
                  _full_spec(128, 256), _full_spec(1, 256)],
        out_specs=(_row_spec(128), _row_spec(128)),
        out_shape=(jax.ShapeDtypeStruct((N, 128), jnp.float32),
                   jax.ShapeDtypeStruct((N, 128), jnp.float32)),
    )(t0p, xp, dinv, W1, b1.reshape(1, 256))


def _tcc_body(ta_ref, tb_ref, ga_ref, gb_ref, dinv_ref, w2_ref, b2_ref,
              w3_ref, g2_ref):
    dinv = dinv_ref[...]
    sa = dinv * (ta_ref[0] + ta_ref[1] + ga_ref[...])
    sb = dinv * (tb_ref[0] + tb_ref[1] + gb_ref[...])
    s1 = jnp.concatenate([sa, sb], axis=1)
    h2 = _leaky(jnp.dot(s1, w2_ref[...], precision=_HIGH,
                        preferred_element_type=jnp.float32) + b2_ref[...])
    g2_ref[...] = dinv * jnp.dot(h2, w3_ref[...], precision=_HIGH,
                                 preferred_element_type=jnp.float32)


def _tc_c(t1ap, t1bp, g1a, g1b, dinv, W2, b2, W3):
    return pl.pallas_call(
        _tcc_body,
        grid=(_G,),
        in_specs=[_part_spec(128), _part_spec(128), _row_spec(128),
                  _row_spec(128), _row_spec(1), _full_spec(256, 256),
                  _full_spec(1, 256), _full_spec(256, 128)],
        out_specs=_row_spec(128),
        out_shape=jax.ShapeDtypeStruct((N, 128), jnp.float32),
    )(t1ap, t1bp, g1a, g1b, dinv, W2, b2.reshape(1, 256), W3)


def _tcd_body(t2_ref, g2_ref, dinv_ref, b3_ref, g3_ref):
    dinv = dinv_ref[...]
    h3 = _leaky(dinv * (t2_ref[0] + t2_ref[1] + g2_ref[...]) + b3_ref[...])
    g3_ref[...] = dinv * h3


def _tc_d(t2p, g2, dinv, b3):
    return pl.pallas_call(
        _tcd_body,
        grid=(_G,),
        in_specs=[_part_spec(128), _row_spec(128), _row_spec(1),
                  _full_spec(1, 128)],
        out_specs=_row_spec(128),
        out_shape=jax.ShapeDtypeStruct((N, 128), jnp.float32),
    )(t2p, g2, dinv, b3.reshape(1, 128))


def _tce_body(t3_ref, g3_ref, dinv_ref, w4_ref, b4_ref, wp_ref, bp_ref,
              out_ref):
    dinv = dinv_ref[...]
    s3 = dinv * (t3_ref[0] + t3_ref[1] + g3_ref[...])
    h4 = _leaky(jnp.dot(s3, w4_ref[...], precision=_HIGH,
                        preferred_element_type=jnp.float32) + b4_ref[...])
    out_ref[...] = jnp.dot(h4, wp_ref[...], precision=_HIGH,
                           preferred_element_type=jnp.float32) + bp_ref[...]


def _tc_e(t3p, g3, dinv, W4, b4, Wp, bp):
    return pl.pallas_call(
        _tce_body,
        grid=(_G,),
        in_specs=[_part_spec(128), _row_spec(128), _row_spec(1),
                  _full_spec(128, 64), _full_spec(1, 64), _full_spec(64, 40),
                  _full_spec(1, 40)],
        out_specs=_row_spec(40),
        out_shape=jax.ShapeDtypeStruct((N, 40), jnp.float32),
    )(t3p, g3, dinv, W4, b4.reshape(1, 64), Wp, bp.reshape(1, 40))


def kernel(x, edge_index, W1, b1, W2, b2, W3, b3, W4, b4, Wp, bp):
    ei = edge_index.astype(jnp.int32)
    pad = E_PAD - E
    srcp = jnp.concatenate([ei[0], jnp.zeros((pad,), jnp.int32)])
    dstp = jnp.concatenate([ei[1], jnp.full((pad,), N, jnp.int32)])
    src3 = srcp.reshape(NT, NB, B)
    dst3 = dstp.reshape(NT, NB, B)
    ones128 = jnp.ones((B, 128), jnp.float32)
    z128 = jnp.zeros((RPT, 128), jnp.float32)

    agg128 = _make_agg(128)

    deg_parts = _deg_kernel(dst3, ones128, z128)
    dinv, xp = _tc_a(deg_parts, x)

    t0p = agg128(xp, src3, dst3, z128)
    g1a, g1b = _tc_b(t0p, xp, dinv, W1, b1)

    t1ap = agg128(g1a, src3, dst3, z128)
    t1bp = agg128(g1b, src3, dst3, z128)
    g2 = _tc_c(t1ap, t1bp, g1a, g1b, dinv, W2, b2, W3)

    t2p = agg128(g2, src3, dst3, z128)
    g3 = _tc_d(t2p, g2, dinv, b3)

    t3p = agg128(g3, src3, dst3, z128)
    return _tc_e(t3p, g3, dinv, W4, b4, Wp, bp)


# speedup vs baseline: 7.0950x; 1.2329x over previous
import functools

import jax
import jax.numpy as jnp
from jax import lax
from jax.experimental import pallas as pl
from jax.experimental.pallas import tpu as pltpu
from jax.experimental.pallas import tpu_sc as plsc

N = 10000
E = 320000
NC, NS, L = 2, 16, 16
NT = NC * NS
B = 128
NB = 80
EPT = NB * B
E_PAD = NT * EPT
PAD_T = EPT - E // NT
NA = N + PAD_T
RPT = 624
REM = N - NS * RPT

_MESH = plsc.VectorSubcoreMesh(core_axis_name="c", subcore_axis_name="s")
_HIGH = jax.lax.Precision.HIGHEST


def _leaky(v):
    return jnp.where(v >= 0, v, 0.01 * v)


def _deg_body(dst_hbm, ones_hbm, zero_hbm, out_hbm, didx_b, ones_v, acc_sh, sem):
    c = lax.axis_index("c")
    s = lax.axis_index("s")
    tid = c * NS + s
    pltpu.sync_copy(ones_hbm, ones_v)
    pltpu.sync_copy(zero_hbm, acc_sh.at[pl.ds(s * RPT, RPT)])

    @pl.when(s == 0)
    def _():
        pltpu.sync_copy(zero_hbm.at[pl.ds(0, REM)],
                        acc_sh.at[pl.ds(NS * RPT, REM)])

    plsc.subcore_barrier()

    def body(i, _):
        pltpu.sync_copy(dst_hbm.at[tid, i], didx_b)
        pltpu.sync_copy(ones_v, acc_sh.at[didx_b], add=True)
        return 0

    lax.fori_loop(0, NB, body, 0)
    plsc.subcore_barrier()
    pltpu.sync_copy(acc_sh.at[pl.ds(s * RPT, RPT)],
                    out_hbm.at[c, pl.ds(s * RPT, RPT)])

    @pl.when(s == 0)
    def _():
        pltpu.sync_copy(acc_sh.at[pl.ds(NS * RPT, REM)],
                        out_hbm.at[c, pl.ds(NS * RPT, REM)])


@functools.partial(
    pl.kernel,
    out_type=jax.ShapeDtypeStruct((NC, N, 128), jnp.float32),
    mesh=_MESH,
    scratch_types=[
        pltpu.VMEM((B,), jnp.int32),
        pltpu.VMEM((B, 128), jnp.float32),
        pltpu.VMEM_SHARED((NA, 128), jnp.float32),
        pltpu.SemaphoreType.DMA,
    ],
)
def _deg_kernel(dst_hbm, ones_hbm, zero_hbm, out_hbm, didx_b, ones_v, acc_sh, sem):
    _deg_body(dst_hbm, ones_hbm, zero_hbm, out_hbm, didx_b, ones_v, acc_sh, sem)


def _agg_body(W, table_hbm, src_hbm, dst_hbm, zero_hbm, out_hbm,
              sidx, didx, rows, acc_sh, isem, gsem, ssem):
    c = lax.axis_index("c")
    s = lax.axis_index("s")
    tid = c * NS + s
    pltpu.sync_copy(zero_hbm, acc_sh.at[pl.ds(s * RPT, RPT)])

    @pl.when(s == 0)
    def _():
        pltpu.sync_copy(zero_hbm.at[pl.ds(0, REM)],
                        acc_sh.at[pl.ds(NS * RPT, REM)])

    for q in range(2):
        pltpu.async_copy(src_hbm.at[tid, q], sidx[q], isem[q])
        pltpu.async_copy(dst_hbm.at[tid, q], didx[q], isem[q])

    plsc.subcore_barrier()

    def body(j, _):
        for p in range(4):
            b = 4 * j + p
            q, r = p, p % 2
            pltpu.make_async_copy(src_hbm.at[tid, 0], sidx[q], isem[q]).wait()
            pltpu.make_async_copy(dst_hbm.at[tid, 0], didx[q], isem[q]).wait()
            if p < 2:
                @pl.when(j > 0)
                def _():
                    pltpu.make_async_copy(
                        rows[r], acc_sh.at[pl.ds(0, B)], ssem[r]).wait()
            else:
                pltpu.make_async_copy(
                    rows[r], acc_sh.at[pl.ds(0, B)], ssem[r]).wait()
            nb = b + 2
            nb = jnp.where(nb < NB, nb, nb - NB)
            nq = (p + 2) % 4
            pltpu.async_copy(src_hbm.at[tid, nb], sidx[nq], isem[nq])
            pltpu.async_copy(dst_hbm.at[tid, nb], didx[nq], isem[nq])
            pltpu.async_copy(table_hbm.at[sidx[q]], rows[r], gsem).wait()
            pltpu.async_copy(rows[r], acc_sh.at[didx[q]], ssem[r], add=True)
        return 0

    lax.fori_loop(0, NB // 4, body, 0)
    for q in range(2):
        pltpu.make_async_copy(src_hbm.at[tid, 0], sidx[q], isem[q]).wait()
        pltpu.make_async_copy(dst_hbm.at[tid, 0], didx[q], isem[q]).wait()
    for r in range(2):
        pltpu.make_async_copy(rows[r], acc_sh.at[pl.ds(0, B)], ssem[r]).wait()
    plsc.subcore_barrier()
    pltpu.sync_copy(acc_sh.at[pl.ds(s * RPT, RPT)],
                    out_hbm.at[c, pl.ds(s * RPT, RPT)])

    @pl.when(s == 0)
    def _():
        pltpu.sync_copy(acc_sh.at[pl.ds(NS * RPT, REM)],
                        out_hbm.at[c, pl.ds(NS * RPT, REM)])


@functools.cache
def _make_agg(W):
    @functools.partial(
        pl.kernel,
        out_type=jax.ShapeDtypeStruct((NC, N, W), jnp.float32),
        mesh=_MESH,
        scratch_types=[
            [pltpu.VMEM((B,), jnp.int32)] * 4,
            [pltpu.VMEM((B,), jnp.int32)] * 4,
            [pltpu.VMEM((B, W), jnp.float32)] * 2,
            pltpu.VMEM_SHARED((NA, W), jnp.float32),
            [pltpu.SemaphoreType.DMA] * 4,
            pltpu.SemaphoreType.DMA,
            [pltpu.SemaphoreType.DMA] * 2,
        ],
    )
    def _agg(table_hbm, src_hbm, dst_hbm, zero_hbm, out_hbm,
             sidx, didx, rows, acc_sh, isem, gsem, ssem):
        _agg_body(W, table_hbm, src_hbm, dst_hbm, zero_hbm, out_hbm,
                  sidx, didx, rows, acc_sh, isem, gsem, ssem)

    return _agg


def _tca_body(dp_ref, x_ref, dinv_ref, xp_ref):
    deg = dp_ref[0, :, 0:1] + dp_ref[1, :, 0:1] + 1.0
    dinv = lax.rsqrt(deg)
    dinv_ref[...] = dinv
    xp_ref[...] = x_ref[...] * dinv


def _tc_a(deg_parts, x):
    return pl.pallas_call(
        _tca_body,
        out_shape=(jax.ShapeDtypeStruct((N, 1), jnp.float32),
                   jax.ShapeDtypeStruct((N, 128), jnp.float32)),
    )(deg_parts, x)


_RB = 1000
_G = N // _RB


def _row_spec(w):
    return pl.BlockSpec((_RB, w), lambda i: (i, 0))


def _part_spec(w):
    return pl.BlockSpec((NC, _RB, w), lambda i: (0, i, 0))


def _full_spec(a, b):
    return pl.BlockSpec((a, b), lambda i: (0, 0))


def _tcb_body(t0_ref, xp_ref, dinv_ref, w1_ref, b1_ref, ga_ref, gb_ref):
    dinv = dinv_ref[...]
    s0 = dinv * (t0_ref[0] + t0_ref[1] + xp_ref[...])
    h1 = _leaky(jnp.dot(s0, w1_ref[...], precision=_HIGH,
                        preferred_element_type=jnp.float32) + b1_ref[...])
    g1 = dinv * h1
    ga_ref[...] = g1[:, :128]
    gb_ref[...] = g1[:, 128:]


def _tc_b(t0p, xp, dinv, W1, b1):
    return pl.pallas_call(
        _tcb_body,
        grid=(_G,),
        in_specs=[_part_spec(128), _row_spec(128), _row_spec(1),
                  _full_spec(128, 256), _full_spec(1, 256)],
        out_specs=(_row_spec(128), _row_spec(128)),
        out_shape=(jax.ShapeDtypeStruct((N, 128), jnp.float32),
                   jax.ShapeDtypeStruct((N, 128), jnp.float32)),
    )(t0p, xp, dinv, W1, b1.reshape(1, 256))


def _tcc_body(ta_ref, tb_ref, ga_ref, gb_ref, dinv_ref, w2_ref, b2_ref,
              w3_ref, g2_ref):
    dinv = dinv_ref[...]
    sa = dinv * (ta_ref[0] + ta_ref[1] + ga_ref[...])
    sb = dinv * (tb_ref[0] + tb_ref[1] + gb_ref[...])
    s1 = jnp.concatenate([sa, sb], axis=1)
    h2 = _leaky(jnp.dot(s1, w2_ref[...], precision=_HIGH,
                        preferred_element_type=jnp.float32) + b2_ref[...])
    g2_ref[...] = dinv * jnp.dot(h2, w3_ref[...], precision=_HIGH,
                                 preferred_element_type=jnp.float32)


def _tc_c(t1ap, t1bp, g1a, g1b, dinv, W2, b2, W3):
    return pl.pallas_call(
        _tcc_body,
        grid=(_G,),
        in_specs=[_part_spec(128), _part_spec(128), _row_spec(128),
                  _row_spec(128), _row_spec(1), _full_spec(256, 256),
                  _full_spec(1, 256), _full_spec(256, 128)],
        out_specs=_row_spec(128),
        out_shape=jax.ShapeDtypeStruct((N, 128), jnp.float32),
    )(t1ap, t1bp, g1a, g1b, dinv, W2, b2.reshape(1, 256), W3)


def _tcd_body(t2_ref, g2_ref, dinv_ref, b3_ref, g3_ref):
    dinv = dinv_ref[...]
    h3 = _leaky(dinv * (t2_ref[0] + t2_ref[1] + g2_ref[...]) + b3_ref[...])
    g3_ref[...] = dinv * h3


def _tc_d(t2p, g2, dinv, b3):
    return pl.pallas_call(
        _tcd_body,
        grid=(_G,),
        in_specs=[_part_spec(128), _row_spec(128), _row_spec(1),
                  _full_spec(1, 128)],
        out_specs=_row_spec(128),
        out_shape=jax.ShapeDtypeStruct((N, 128), jnp.float32),
    )(t2p, g2, dinv, b3.reshape(1, 128))


def _tce_body(t3_ref, g3_ref, dinv_ref, w4_ref, b4_ref, wp_ref, bp_ref,
              out_ref):
    dinv = dinv_ref[...]
    s3 = dinv * (t3_ref[0] + t3_ref[1] + g3_ref[...])
    h4 = _leaky(jnp.dot(s3, w4_ref[...], precision=_HIGH,
                        preferred_element_type=jnp.float32) + b4_ref[...])
    out_ref[...] = jnp.dot(h4, wp_ref[...], precision=_HIGH,
                           preferred_element_type=jnp.float32) + bp_ref[...]


def _tc_e(t3p, g3, dinv, W4, b4, Wp, bp):
    return pl.pallas_call(
        _tce_body,
        grid=(_G,),
        in_specs=[_part_spec(128), _row_spec(128), _row_spec(1),
                  _full_spec(128, 64), _full_spec(1, 64), _full_spec(64, 40),
                  _full_spec(1, 40)],
        out_specs=_row_spec(40),
        out_shape=jax.ShapeDtypeStruct((N, 40), jnp.float32),
    )(t3p, g3, dinv, W4, b4.reshape(1, 64), Wp, bp.reshape(1, 40))


def kernel(x, edge_index, W1, b1, W2, b2, W3, b3, W4, b4, Wp, bp):
    ei = edge_index.astype(jnp.int32)
    zpad = jnp.zeros((NT, PAD_T), jnp.int32)
    tpad = jnp.broadcast_to(N + jnp.arange(PAD_T, dtype=jnp.int32), (NT, PAD_T))
    srcp = jnp.concatenate([ei[0].reshape(NT, E // NT), zpad], axis=1)
    dstp = jnp.concatenate([ei[1].reshape(NT, E // NT), tpad], axis=1)
    src3 = srcp.reshape(NT, NB, B)
    dst3 = dstp.reshape(NT, NB, B)
    ones128 = jnp.ones((B, 128), jnp.float32)
    z128 = jnp.zeros((RPT, 128), jnp.float32)

    agg128 = _make_agg(128)

    deg_parts = _deg_kernel(dst3, ones128, z128)
    dinv, xp = _tc_a(deg_parts, x)

    t0p = agg128(xp, src3, dst3, z128)
    g1a, g1b = _tc_b(t0p, xp, dinv, W1, b1)

    t1ap = agg128(g1a, src3, dst3, z128)
    t1bp = agg128(g1b, src3, dst3, z128)
    g2 = _tc_c(t1ap, t1bp, g1a, g1b, dinv, W2, b2, W3)

    t2p = agg128(g2, src3, dst3, z128)
    g3 = _tc_d(t2p, g2, dinv, b3)

    t3p = agg128(g3, src3, dst3, z128)
    return _tc_e(t3p, g3, dinv, W4, b4, Wp, bp)
